# baseline (speedup 1.0000x reference)
import jax
import jax.numpy as jnp
from jax.experimental import pallas as pl
from jax.experimental.pallas import tpu as pltpu


def _make_body(B, S, D, R, K):
    C = (B * S) // R

    def body(xf_hbm, pe2_hbm, out_hbm, *rest):
        pe_vmem = rest[0]
        x_bufs = rest[1:1 + K]
        pe_sem = rest[1 + K]
        in_sem = rest[2 + K]
        out_sem = rest[3 + K]

        def in_copy(c):
            return pltpu.make_async_copy(
                xf_hbm.at[pl.ds(c * R, R), :], x_bufs[c % K], in_sem.at[c % K])

        def out_copy(c):
            return pltpu.make_async_copy(
                x_bufs[c % K], out_hbm.at[pl.ds(c * R, R), :], out_sem.at[c % K])

        pe_copy = pltpu.make_async_copy(
            pe2_hbm.at[:, pl.ds(0, D)], pe_vmem, pe_sem)
        pe_copy.start()
        for k in range(min(K, C)):
            in_copy(k).start()
        pe_copy.wait()

        for c in range(C):
            slot = c % K
            in_copy(c).wait()
            if c >= K:
                out_copy(c - K).wait()
            smod = (c * R) % S
            x_bufs[slot][...] = x_bufs[slot][...] + pe_vmem[pl.ds(smod, R), :]
            out_copy(c).start()
            if c + K < C:
                in_copy(c + K).start()
        for c in range(max(C - K, 0), C):
            out_copy(c).wait()

    return body


def kernel(x, pe):
    B, S, D = x.shape
    max_seq = pe.shape[1]
    stride = max_seq // S
    pe2 = pe[:, : S * stride, :].reshape(S, stride * D)
    xf = x.reshape(B * S, D)

    R = 512
    K = 4

    out = pl.pallas_call(
        _make_body(B, S, D, R, K),
        in_specs=[
            pl.BlockSpec(memory_space=pl.ANY),
            pl.BlockSpec(memory_space=pl.ANY),
        ],
        out_specs=pl.BlockSpec(memory_space=pl.ANY),
        out_shape=jax.ShapeDtypeStruct((B * S, D), x.dtype),
        scratch_shapes=(
            [pltpu.VMEM((S, D), x.dtype)]
            + [pltpu.VMEM((R, D), x.dtype) for _ in range(K)]
            + [pltpu.SemaphoreType.DMA,
               pltpu.SemaphoreType.DMA((K,)),
               pltpu.SemaphoreType.DMA((K,))]
        ),
    )(xf, pe2)
    return out.reshape(B, S, D)
